# baseline (speedup 1.0000x reference)
import functools

import jax
import jax.numpy as jnp
from jax import lax
from jax.experimental import pallas as pl
from jax.experimental.pallas import tpu as pltpu
from jax.experimental.pallas import tpu_sc as plsc

B, T = 4, 8192
N_TOK = B * T
D_MODEL = 1024
SPD = 86400
PERIODS = (SPD, SPD // 2, SPD // 3, SPD // 4, SPD // 6)
NT = len(PERIODS)
SUB = 204
SEG = 256
K_TOT = NT * SEG
LANES = 16
NC, NS = 2, 16
NW = NC * NS
TOK_W = N_TOK // NW
CHUNK = 128
NCHUNK = TOK_W // CHUNK


def _mod_period(v, period):
    out = v
    k = period
    while k < SPD:
        out = out - jnp.where(v >= k, jnp.int32(period), jnp.int32(0))
        k += period
    return out


def _sc_gather_one(t_flat, table_packed, period):
    mesh = plsc.VectorSubcoreMesh(core_axis_name="c", subcore_axis_name="s")
    out_type = jax.ShapeDtypeStruct((N_TOK, 128), jnp.int32)
    scratch = (
        [pltpu.VMEM((TOK_W,), jnp.int32),
         pltpu.VMEM((TOK_W,), jnp.int32)]
        + [pltpu.VMEM((CHUNK, 128), jnp.int32) for _ in range(2)]
        + [pltpu.SemaphoreType.DMA for _ in range(2)]
    )

    @functools.partial(pl.kernel, mesh=mesh, out_type=out_type,
                       scratch_types=scratch)
    def k(t_hbm, tbl, out, tok_v, idx_v, b0, b1, sm0, sm1):
        wid = lax.axis_index("s") * NC + lax.axis_index("c")
        base = wid * TOK_W
        pltpu.sync_copy(t_hbm.at[pl.ds(base, TOK_W)], tok_v)

        def mod_body(c, carry):
            off = c * LANES
            idx_v[pl.ds(off, LANES)] = _mod_period(
                tok_v[pl.ds(off, LANES)], period)
            return carry

        lax.fori_loop(0, TOK_W // LANES, mod_body, 0)

        def pair_body(c, carry):
            ch0 = c * 2
            ch1 = ch0 + 1
            g0 = pltpu.async_copy(
                tbl.at[idx_v.at[pl.ds(ch0 * CHUNK, CHUNK)]], b0, sm0)
            g1 = pltpu.async_copy(
                tbl.at[idx_v.at[pl.ds(ch1 * CHUNK, CHUNK)]], b1, sm1)
            g0.wait()
            pltpu.sync_copy(b0, out.at[pl.ds(base + ch0 * CHUNK, CHUNK)])
            g1.wait()
            pltpu.sync_copy(b1, out.at[pl.ds(base + ch1 * CHUNK, CHUNK)])
            return carry

        lax.fori_loop(0, NCHUNK // 2, pair_body, 0)

    return k(t_flat, table_packed)


def _tc_project(embs, w_even, w_odd, bias):
    BM = 256
    ne = len(embs)
    HW = SEG // 2

    def body(*refs):
        e_refs = refs[:ne]
        we_ref = refs[ne]
        wo_ref = refs[ne + 1]
        b_ref = refs[ne + 2]
        o_ref = refs[ne + 3]
        acc = b_ref[...]
        for i, e_ref in enumerate(e_refs):
            e = e_ref[...]
            lo = jax.lax.bitcast_convert_type(
                e << 16, jnp.float32).astype(jnp.bfloat16)
            hi = jax.lax.bitcast_convert_type(
                e & jnp.int32(-65536), jnp.float32).astype(jnp.bfloat16)
            acc = acc + jnp.dot(lo, we_ref[i * HW:(i + 1) * HW],
                                preferred_element_type=jnp.float32)
            acc = acc + jnp.dot(hi, wo_ref[i * HW:(i + 1) * HW],
                                preferred_element_type=jnp.float32)
        o_ref[...] = acc

    in_specs = (
        [pl.BlockSpec((BM, 128), lambda m: (m, 0)) for _ in range(ne)]
        + [pl.BlockSpec((K_TOT // 2, D_MODEL), lambda m: (0, 0)),
           pl.BlockSpec((K_TOT // 2, D_MODEL), lambda m: (0, 0)),
           pl.BlockSpec((1, D_MODEL), lambda m: (0, 0))]
    )
    return pl.pallas_call(
        body,
        grid=(N_TOK // BM,),
        in_specs=in_specs,
        out_specs=pl.BlockSpec((BM, D_MODEL), lambda m: (m, 0)),
        out_shape=jax.ShapeDtypeStruct((N_TOK, D_MODEL), jnp.float32),
    )(*embs, w_even, w_odd, bias)


def kernel(x, time_indices, table0, table1, table2, table3, table4, Wp, bp):
    del x
    t_flat = time_indices.reshape(N_TOK).astype(jnp.int32)
    tables = (table0, table1, table2, table3, table4)
    embs = []
    for i, tbl in enumerate(tables):
        bf = jnp.pad(tbl, ((0, 0), (0, SEG - SUB))).astype(jnp.bfloat16)
        packed = jax.lax.bitcast_convert_type(
            bf.reshape(PERIODS[i], 128, 2), jnp.int32)
        embs.append(_sc_gather_one(t_flat, packed, PERIODS[i]))
    zrows = jnp.zeros((SEG - SUB, D_MODEL), jnp.float32)
    w = jnp.concatenate(
        [p for i in range(NT)
         for p in (Wp[i * SUB:(i + 1) * SUB], zrows)]).astype(jnp.bfloat16)
    out = _tc_project(embs, w[0::2], w[1::2], bp.reshape(1, D_MODEL))
    return out.reshape(B, T, D_MODEL)
